# baseline (speedup 1.0000x reference)
import functools

import jax
import jax.numpy as jnp
from jax.experimental import pallas as pl
from jax.experimental.pallas import tpu as pltpu

_TM = 1024
_TN = 512


def _addmm_kernel(i_ref, x_hbm, y_ref, o_ref, xbuf, sems, *, beta, alpha, nm):
    m = pl.program_id(0)
    n = pl.program_id(1)
    slot = jax.lax.rem(m, 2)

    def start_copy(mi, s):
        pltpu.make_async_copy(
            x_hbm.at[pl.ds(mi * _TM, _TM), :], xbuf.at[s], sems.at[s]
        ).start()

    @pl.when((m == 0) & (n == 0))
    def _():
        start_copy(0, 0)
        start_copy(1, 1)

    @pl.when(n == 0)
    def _():
        pltpu.make_async_copy(
            x_hbm.at[pl.ds(0, _TM), :], xbuf.at[slot], sems.at[slot]
        ).wait()

    @pl.when((n == 1) & (m >= 1) & (m + 1 < nm))
    def _():
        start_copy(m + 1, 1 - slot)

    acc = jax.lax.dot_general(
        xbuf[slot],
        y_ref[...],
        (((1,), (0,)), ((), ())),
        preferred_element_type=jnp.float32,
    )
    o_ref[...] = beta * i_ref[...] + alpha * acc


def kernel(i, x, y):
    beta, alpha = 1.0, 2.0
    M, K = x.shape
    _, N = y.shape
    i2 = i.reshape(1, N)
    yb = y.astype(jnp.bfloat16)

    kfn = functools.partial(_addmm_kernel, beta=beta, alpha=alpha, nm=M // _TM)
    return pl.pallas_call(
        kfn,
        out_shape=jax.ShapeDtypeStruct((M, N), jnp.float32),
        grid=(M // _TM, N // _TN),
        in_specs=[
            pl.BlockSpec((1, _TN), lambda m, n: (0, n)),
            pl.BlockSpec(memory_space=pl.ANY),
            pl.BlockSpec((K, _TN), lambda m, n: (0, n)),
        ],
        out_specs=pl.BlockSpec((_TM, _TN), lambda m, n: (m, n)),
        scratch_shapes=[
            pltpu.VMEM((2, _TM, K), jnp.float32),
            pltpu.SemaphoreType.DMA((2,)),
        ],
        compiler_params=pltpu.CompilerParams(
            dimension_semantics=("arbitrary", "arbitrary")
        ),
    )(i2, x, yb)


# speedup vs baseline: 1.1381x; 1.1381x over previous
import functools

import jax
import jax.numpy as jnp
from jax.experimental import pallas as pl
from jax.experimental.pallas import tpu as pltpu

_TM = 1024
_TN = 512


def _addmm_kernel(i_ref, x_hbm, y_ref, o_ref, xbuf, sems, *, beta, alpha, nm):
    m = pl.program_id(0)
    n = pl.program_id(1)
    slot = jax.lax.rem(m, 2)

    def start_copy(mi, s):
        pltpu.make_async_copy(
            x_hbm.at[pl.ds(mi * _TM, _TM), :], xbuf.at[s], sems.at[s]
        ).start()

    @pl.when((m == 0) & (n == 0))
    def _():
        start_copy(0, 0)
        start_copy(1, 1)

    @pl.when(n == 0)
    def _():
        pltpu.make_async_copy(
            x_hbm.at[pl.ds(0, _TM), :], xbuf.at[slot], sems.at[slot]
        ).wait()

    @pl.when((n == 1) & (m >= 1) & (m + 1 < nm))
    def _():
        start_copy(m + 1, 1 - slot)

    acc = jnp.dot(xbuf[slot], y_ref[...], preferred_element_type=jnp.float32)
    o_ref[...] = beta * i_ref[...] + alpha * acc


def kernel(i, x, y):
    beta, alpha = 1.0, 2.0
    M, K = x.shape
    _, N = y.shape
    i2 = i.reshape(1, N)

    kfn = functools.partial(_addmm_kernel, beta=beta, alpha=alpha, nm=M // _TM)
    return pl.pallas_call(
        kfn,
        out_shape=jax.ShapeDtypeStruct((M, N), jnp.float32),
        grid=(M // _TM, N // _TN),
        in_specs=[
            pl.BlockSpec((1, _TN), lambda m, n: (0, n)),
            pl.BlockSpec(memory_space=pl.ANY),
            pl.BlockSpec((K, _TN), lambda m, n: (0, n)),
        ],
        out_specs=pl.BlockSpec((_TM, _TN), lambda m, n: (m, n)),
        scratch_shapes=[
            pltpu.VMEM((2, _TM, K), jnp.float32),
            pltpu.SemaphoreType.DMA((2,)),
        ],
        compiler_params=pltpu.CompilerParams(
            dimension_semantics=("arbitrary", "arbitrary")
        ),
    )(i2, x, y)
